# Initial kernel scaffold; baseline (speedup 1.0000x reference)
import jax
import jax.numpy as jnp
from jax.experimental import pallas as pl


def kernel(x, edge_index, Wq, bq, Wk, bk, Wv, bv):
    raise NotImplementedError("write your pallas kernel here")


# speedup vs baseline: 11.2752x; 11.2752x over previous
import functools

import jax
import jax.numpy as jnp
from jax import lax
from jax.experimental import pallas as pl
from jax.experimental.pallas import tpu as pltpu
from jax.experimental.pallas import tpu_sc as plsc

N = 10000
HD = 128
H = 8
D = 16
E = 320000
NW = 32
E_PER = E // NW
B = 16
NCH = E_PER // B
N_PAD = 10240
SUB_ROWS = N_PAD // 16
ZROWS = 16

f32 = jnp.float32
i32 = jnp.int32


def _mesh():
  return plsc.VectorSubcoreMesh(core_axis_name="c", subcore_axis_name="s")


_SC_PARAMS = pltpu.CompilerParams(needs_layout_passes=False)


def _qkv_body(xr, wqr, bqr, wkr, bkr, wvr, bvr, qr, kr, vr):
  xb = xr[...]
  dn = (((1,), (1,)), ((), ()))
  qr[...] = (lax.dot_general(xb, wqr[...], dn, preferred_element_type=f32)
             + bqr[...]) * 0.25
  kr[...] = lax.dot_general(xb, wkr[...], dn, preferred_element_type=f32) + bkr[...]
  vr[...] = lax.dot_general(xb, wvr[...], dn, preferred_element_type=f32) + bvr[...]


def _qkv(x, Wq, bq, Wk, bk, Wv, bv):
  blk = 2000
  xspec = pl.BlockSpec((blk, HD), lambda i: (i, 0))
  wspec = pl.BlockSpec((HD, HD), lambda i: (0, 0))
  bspec = pl.BlockSpec((1, HD), lambda i: (0, 0))
  ospec = pl.BlockSpec((blk, HD), lambda i: (i, 0))
  return pl.pallas_call(
      _qkv_body,
      grid=(N // blk,),
      in_specs=[xspec, wspec, bspec, wspec, bspec, wspec, bspec],
      out_specs=[ospec, ospec, ospec],
      out_shape=[jax.ShapeDtypeStruct((N, HD), f32)] * 3,
  )(x, Wq, bq.reshape(1, HD), Wk, bk.reshape(1, HD), Wv, bv.reshape(1, HD))


def _score_body(qs, kh, src, dst, score_o, tmax_o,
                src_v, dst_v, kg, qg, sbuf, tm_v, sem):
  c = lax.axis_index("c")
  s = lax.axis_index("s")
  wid = c * 16 + s
  base0 = wid * E_PER
  lane = lax.iota(i32, 16)
  masks = [lane == h for h in range(H)]

  def chunk(ci, rm):
    base = base0 + ci * B
    pltpu.sync_copy(src.at[pl.ds(base, B)], src_v)
    pltpu.sync_copy(dst.at[pl.ds(base, B)], dst_v)
    pltpu.async_copy(kh.at[src_v], kg, sem).wait()
    pltpu.async_copy(qs.at[dst_v], qg, sem).wait()

    def edge(e, rm):
      row = jnp.zeros((16,), f32)
      for h in range(H):
        kv = kg[e, pl.ds(h * D, D)]
        qv = qg[e, pl.ds(h * D, D)]
        sh = jnp.sum(kv * qv)
        row = jnp.where(masks[h], sh, row)
      sbuf[e, :] = row
      return jnp.maximum(rm, row)

    rm = lax.fori_loop(0, B, edge, rm)
    pltpu.sync_copy(sbuf, score_o.at[pl.ds(base, B)])
    return rm

  rm = lax.fori_loop(0, NCH, chunk, jnp.full((16,), -jnp.inf, f32))
  for r in range(8):
    tm_v[r, :] = rm
  pltpu.sync_copy(tm_v, tmax_o.at[wid])


def _score_call(qs, kh, src, dst):
  return pl.kernel(
      _score_body,
      out_type=(jax.ShapeDtypeStruct((E, 16), f32),
                jax.ShapeDtypeStruct((NW, 8, 16), f32)),
      mesh=_mesh(),
      compiler_params=_SC_PARAMS,
      scratch_types=[
          pltpu.VMEM((B,), i32),
          pltpu.VMEM((B,), i32),
          pltpu.VMEM((B, HD), f32),
          pltpu.VMEM((B, HD), f32),
          pltpu.VMEM((B, 16), f32),
          pltpu.VMEM((8, 16), f32),
          pltpu.SemaphoreType.DMA,
      ],
  )(qs, kh, src, dst)


def _global_max(tmax, tm_v):
  def mred(t, m):
    pltpu.sync_copy(tmax.at[t], tm_v)
    return jnp.maximum(m, tm_v[0, :])

  return lax.fori_loop(0, NW, mred, jnp.full((16,), -jnp.inf, f32))


def _agg_body(score, tmax, vh, src, dst, outv, outs,
              src_v, dst_v, idx_v, srow, vg, accv):
  c = lax.axis_index("c")
  s = lax.axis_index("s")
  wid = c * 16 + s
  base0 = wid * E_PER
  lane = lax.iota(i32, 16)
  masks = [lane == h for h in range(H)]
  off = s * SUB_ROWS

  def zero_vg(i, _):
    for j in range(HD // 16):
      vg[i, pl.ds(j * 16, 16)] = jnp.zeros((16,), f32)
    return 0

  def zero_acc():
    lax.fori_loop(0, ZROWS, zero_vg, 0)
    for j in range(SUB_ROWS // ZROWS):
      idx_v[...] = lane + (off + j * ZROWS)
      pltpu.sync_copy(vg, accv.at[idx_v])

  def dump_acc(out_ref):
    for j in range(SUB_ROWS // ZROWS):
      o2 = off + j * ZROWS
      idx_v[...] = lane + o2
      pltpu.sync_copy(accv.at[idx_v], vg)
      pltpu.sync_copy(vg, out_ref.at[c, pl.ds(o2, ZROWS)])

  zero_acc()
  plsc.subcore_barrier()

  def mred(t, m):
    pltpu.sync_copy(tmax.at[t], srow.at[pl.ds(0, 8)])
    return jnp.maximum(m, srow[0, :])

  m = lax.fori_loop(0, NW, mred, jnp.full((16,), -jnp.inf, f32))

  def chunk1(ci, _):
    base = base0 + ci * B
    pltpu.sync_copy(src.at[pl.ds(base, B)], src_v)
    pltpu.sync_copy(dst.at[pl.ds(base, B)], dst_v)
    pltpu.sync_copy(vh.at[src_v], vg)
    pltpu.sync_copy(score.at[pl.ds(base, B)], srow)

    def edge(e, _):
      prow = jnp.exp(srow[e, :] - m)
      for h in range(H):
        ah = jnp.sum(jnp.where(masks[h], prow, 0.0))
        vg[e, pl.ds(h * D, D)] = vg[e, pl.ds(h * D, D)] * ah
      return 0

    lax.fori_loop(0, B, edge, 0)
    pltpu.sync_copy(vg, accv.at[dst_v], add=True)
    return 0

  lax.fori_loop(0, NCH, chunk1, 0)
  plsc.subcore_barrier()
  dump_acc(outv)
  plsc.subcore_barrier()

  zero_acc()
  plsc.subcore_barrier()

  def chunk2(ci, _):
    base = base0 + ci * B
    pltpu.sync_copy(dst.at[pl.ds(base, B)], dst_v)
    pltpu.sync_copy(score.at[pl.ds(base, B)], srow)

    def edge(e, _):
      vg[e, pl.ds(0, 16)] = jnp.exp(srow[e, :] - m)
      return 0

    lax.fori_loop(0, B, edge, 0)
    pltpu.sync_copy(vg, accv.at[dst_v], add=True)
    return 0

  lax.fori_loop(0, ZROWS, zero_vg, 0)
  lax.fori_loop(0, NCH, chunk2, 0)
  plsc.subcore_barrier()
  dump_acc(outs)


def _agg_call(score, tmax, vh, src, dst):
  return pl.kernel(
      _agg_body,
      out_type=(jax.ShapeDtypeStruct((2, N_PAD, HD), f32),
                jax.ShapeDtypeStruct((2, N_PAD, HD), f32)),
      mesh=_mesh(),
      compiler_params=_SC_PARAMS,
      scratch_types=[
          pltpu.VMEM((B,), i32),
          pltpu.VMEM((B,), i32),
          pltpu.VMEM((ZROWS,), i32),
          pltpu.VMEM((B, 16), f32),
          pltpu.VMEM((B, HD), f32),
          pltpu.VMEM_SHARED((N_PAD, HD), f32),
      ],
  )(score, tmax, vh, src, dst)


def _norm_body(v0r, v1r, s0r, s1r, orf):
  blk = orf.shape[0]
  ssum = s0r[0][:, :H] + s1r[0][:, :H] + 1e-16
  inv = 1.0 / ssum
  invb = lax.broadcast_in_dim(inv, (blk, H, D), (0, 1)).reshape(blk, HD)
  orf[...] = (v0r[0] + v1r[0]) * invb


def _normalize(outv, outs):
  blk = 2000
  v0spec = pl.BlockSpec((1, blk, HD), lambda i: (0, i, 0))
  v1spec = pl.BlockSpec((1, blk, HD), lambda i: (1, i, 0))
  ospec = pl.BlockSpec((blk, HD), lambda i: (i, 0))
  return pl.pallas_call(
      _norm_body,
      grid=(N // blk,),
      in_specs=[v0spec, v1spec, v0spec, v1spec],
      out_specs=ospec,
      out_shape=jax.ShapeDtypeStruct((N, HD), f32),
  )(outv, outv, outs, outs)


@jax.jit
def kernel(x, edge_index, Wq, bq, Wk, bk, Wv, bv):
  src = edge_index[0].astype(i32)
  dst = edge_index[1].astype(i32)
  qs, kh, vh = _qkv(x, Wq, bq, Wk, bk, Wv, bv)
  score, tmax = _score_call(qs, kh, src, dst)
  outv, outs = _agg_call(score, tmax, vh, src, dst)
  wv = _normalize(outv, outs)
  return wv.reshape(N, H, D)


# speedup vs baseline: 29.2323x; 2.5926x over previous
import functools

import jax
import jax.numpy as jnp
from jax import lax
from jax.experimental import pallas as pl
from jax.experimental.pallas import tpu as pltpu
from jax.experimental.pallas import tpu_sc as plsc

N = 10000
HD = 128
H = 8
D = 16
E = 320000
NW = 32
E_PER = E // NW
B = 80
NCH = E_PER // B
N_PAD = 10240
SUB_ROWS = N_PAD // 16
ZROWS = 16

f32 = jnp.float32
i32 = jnp.int32


def _mesh():
  return plsc.VectorSubcoreMesh(core_axis_name="c", subcore_axis_name="s")


_SC_PARAMS = pltpu.CompilerParams(needs_layout_passes=False)


def _qkv_body(xr, wqr, bqr, wkr, bkr, wvr, bvr, qr, kr, vr):
  xb = xr[...]
  dn = (((1,), (1,)), ((), ()))
  qr[...] = (lax.dot_general(xb, wqr[...], dn, preferred_element_type=f32)
             + bqr[...]) * 0.25
  kr[...] = lax.dot_general(xb, wkr[...], dn, preferred_element_type=f32) + bkr[...]
  vr[...] = lax.dot_general(xb, wvr[...], dn, preferred_element_type=f32) + bvr[...]


def _qkv(x, Wq, bq, Wk, bk, Wv, bv):
  blk = 2000
  xspec = pl.BlockSpec((blk, HD), lambda i: (i, 0))
  wspec = pl.BlockSpec((HD, HD), lambda i: (0, 0))
  bspec = pl.BlockSpec((1, HD), lambda i: (0, 0))
  ospec = pl.BlockSpec((blk, HD), lambda i: (i, 0))
  return pl.pallas_call(
      _qkv_body,
      grid=(N // blk,),
      in_specs=[xspec, wspec, bspec, wspec, bspec, wspec, bspec],
      out_specs=[ospec, ospec, ospec],
      out_shape=[jax.ShapeDtypeStruct((N, HD), f32)] * 3,
  )(x, Wq, bq.reshape(1, HD), Wk, bk.reshape(1, HD), Wv, bv.reshape(1, HD))


def _score_body(qs, kh, src, dst, score_o, tmax_o,
                src_v, dst_v, kg, qg, sbuf, tm_v, sem):
  c = lax.axis_index("c")
  s = lax.axis_index("s")
  wid = c * 16 + s
  base0 = wid * E_PER
  lane = lax.iota(i32, 16)
  masks = [lane == h for h in range(H)]

  def chunk(ci, rm):
    base = base0 + ci * B
    pltpu.sync_copy(src.at[pl.ds(base, B)], src_v)
    pltpu.sync_copy(dst.at[pl.ds(base, B)], dst_v)
    pltpu.async_copy(kh.at[src_v], kg, sem).wait()
    pltpu.async_copy(qs.at[dst_v], qg, sem).wait()

    def edge(e, rm):
      row = jnp.zeros((16,), f32)
      for h in range(H):
        kv = kg[e, pl.ds(h * D, D)]
        qv = qg[e, pl.ds(h * D, D)]
        sh = jnp.sum(kv * qv)
        row = jnp.where(masks[h], sh, row)
      sbuf[e, :] = row
      return jnp.maximum(rm, row)

    rm = lax.fori_loop(0, B, edge, rm)
    pltpu.sync_copy(sbuf, score_o.at[pl.ds(base, B)])
    return rm

  rm = lax.fori_loop(0, NCH, chunk, jnp.full((16,), -jnp.inf, f32))
  for r in range(8):
    tm_v[r, :] = rm
  pltpu.sync_copy(tm_v, tmax_o.at[wid])


def _score_call(qs, kh, src, dst):
  return pl.kernel(
      _score_body,
      out_type=(jax.ShapeDtypeStruct((E, 16), f32),
                jax.ShapeDtypeStruct((NW, 8, 16), f32)),
      mesh=_mesh(),
      compiler_params=_SC_PARAMS,
      scratch_types=[
          pltpu.VMEM((B,), i32),
          pltpu.VMEM((B,), i32),
          pltpu.VMEM((B, HD), f32),
          pltpu.VMEM((B, HD), f32),
          pltpu.VMEM((B, 16), f32),
          pltpu.VMEM((8, 16), f32),
          pltpu.SemaphoreType.DMA,
      ],
  )(qs, kh, src, dst)


def _global_max(tmax, tm_v):
  def mred(t, m):
    pltpu.sync_copy(tmax.at[t], tm_v)
    return jnp.maximum(m, tm_v[0, :])

  return lax.fori_loop(0, NW, mred, jnp.full((16,), -jnp.inf, f32))


def _agg_body(score, tmax, vh, src, dst, outv, outs,
              src_v, dst_v, idx_v, srow, vg, zbuf, accv):
  c = lax.axis_index("c")
  s = lax.axis_index("s")
  wid = c * 16 + s
  base0 = wid * E_PER
  lane = lax.iota(i32, 16)
  masks = [lane == h for h in range(H)]
  off = s * SUB_ROWS

  def zero_zbuf(i, _):
    for j in range(HD // 16):
      zbuf[i, pl.ds(j * 16, 16)] = jnp.zeros((16,), f32)
    return 0

  def zero_acc():
    lax.fori_loop(0, ZROWS, zero_zbuf, 0)
    for j in range(SUB_ROWS // ZROWS):
      idx_v[...] = lane + (off + j * ZROWS)
      pltpu.sync_copy(zbuf, accv.at[idx_v])

  def dump_acc(out_ref):
    for j in range(SUB_ROWS // ZROWS):
      o2 = off + j * ZROWS
      idx_v[...] = lane + o2
      pltpu.sync_copy(accv.at[idx_v], zbuf)
      pltpu.sync_copy(zbuf, out_ref.at[c, pl.ds(o2, ZROWS)])

  zero_acc()
  plsc.subcore_barrier()

  def mred(t, m):
    pltpu.sync_copy(tmax.at[t], srow.at[pl.ds(0, 8)])
    return jnp.maximum(m, srow[0, :])

  m = lax.fori_loop(0, NW, mred, jnp.full((16,), -jnp.inf, f32))

  def chunk1(ci, _):
    base = base0 + ci * B
    pltpu.sync_copy(src.at[pl.ds(base, B)], src_v)
    pltpu.sync_copy(dst.at[pl.ds(base, B)], dst_v)
    pltpu.sync_copy(vh.at[src_v], vg)
    pltpu.sync_copy(score.at[pl.ds(base, B)], srow)

    def edge(e, _):
      prow = jnp.exp(srow[e, :] - m)
      for h in range(H):
        ah = jnp.sum(jnp.where(masks[h], prow, 0.0))
        vg[e, pl.ds(h * D, D)] = vg[e, pl.ds(h * D, D)] * ah
      return 0

    lax.fori_loop(0, B, edge, 0)
    pltpu.sync_copy(vg, accv.at[dst_v], add=True)
    return 0

  lax.fori_loop(0, NCH, chunk1, 0)
  plsc.subcore_barrier()
  dump_acc(outv)
  plsc.subcore_barrier()

  zero_acc()
  plsc.subcore_barrier()

  def chunk2(ci, _):
    base = base0 + ci * B
    pltpu.sync_copy(dst.at[pl.ds(base, B)], dst_v)
    pltpu.sync_copy(score.at[pl.ds(base, B)], srow)

    def edge(e, _):
      vg[e, pl.ds(0, 16)] = jnp.exp(srow[e, :] - m)
      return 0

    lax.fori_loop(0, B, edge, 0)
    pltpu.sync_copy(vg, accv.at[dst_v], add=True)
    return 0

  def zero_vg(i, _):
    for j in range(HD // 16):
      vg[i, pl.ds(j * 16, 16)] = jnp.zeros((16,), f32)
    return 0

  lax.fori_loop(0, B, zero_vg, 0)
  lax.fori_loop(0, NCH, chunk2, 0)
  plsc.subcore_barrier()
  dump_acc(outs)


def _agg_call(score, tmax, vh, src, dst):
  return pl.kernel(
      _agg_body,
      out_type=(jax.ShapeDtypeStruct((2, N_PAD, HD), f32),
                jax.ShapeDtypeStruct((2, N_PAD, HD), f32)),
      mesh=_mesh(),
      compiler_params=_SC_PARAMS,
      scratch_types=[
          pltpu.VMEM((B,), i32),
          pltpu.VMEM((B,), i32),
          pltpu.VMEM((ZROWS,), i32),
          pltpu.VMEM((B, 16), f32),
          pltpu.VMEM((B, HD), f32),
          pltpu.VMEM((ZROWS, HD), f32),
          pltpu.VMEM_SHARED((N_PAD, HD), f32),
      ],
  )(score, tmax, vh, src, dst)


def _norm_body(v0r, v1r, s0r, s1r, orf):
  blk = orf.shape[0]
  ssum = s0r[0][:, :H] + s1r[0][:, :H] + 1e-16
  inv = 1.0 / ssum
  invb = lax.broadcast_in_dim(inv, (blk, H, D), (0, 1)).reshape(blk, HD)
  orf[...] = (v0r[0] + v1r[0]) * invb


def _normalize(outv, outs):
  blk = 2000
  v0spec = pl.BlockSpec((1, blk, HD), lambda i: (0, i, 0))
  v1spec = pl.BlockSpec((1, blk, HD), lambda i: (1, i, 0))
  ospec = pl.BlockSpec((blk, HD), lambda i: (i, 0))
  return pl.pallas_call(
      _norm_body,
      grid=(N // blk,),
      in_specs=[v0spec, v1spec, v0spec, v1spec],
      out_specs=ospec,
      out_shape=jax.ShapeDtypeStruct((N, HD), f32),
  )(outv, outv, outs, outs)


@jax.jit
def kernel(x, edge_index, Wq, bq, Wk, bk, Wv, bv):
  src = edge_index[0].astype(i32)
  dst = edge_index[1].astype(i32)
  qs, kh, vh = _qkv(x, Wq, bq, Wk, bk, Wv, bv)
  score, tmax = _score_call(qs, kh, src, dst)
  outv, outs = _agg_call(score, tmax, vh, src, dst)
  wv = _normalize(outv, outs)
  return wv.reshape(N, H, D)


# speedup vs baseline: 38.3846x; 1.3131x over previous
import functools

import jax
import jax.numpy as jnp
from jax import lax
from jax.experimental import pallas as pl
from jax.experimental.pallas import tpu as pltpu
from jax.experimental.pallas import tpu_sc as plsc

N = 10000
HD = 128
H = 8
D = 16
E = 320000
NW = 32
E_PER = E // NW
B = 40
NCH = E_PER // B
N_PAD = 10240
SUB_ROWS = N_PAD // 16
ZROWS = 16

f32 = jnp.float32
i32 = jnp.int32


def _mesh():
  return plsc.VectorSubcoreMesh(core_axis_name="c", subcore_axis_name="s")


_SC_PARAMS = pltpu.CompilerParams(needs_layout_passes=False)


def _qkv_body(xr, wqr, bqr, wkr, bkr, wvr, bvr, qr, kr, vr):
  xb = xr[...]
  dn = (((1,), (1,)), ((), ()))
  qr[...] = (lax.dot_general(xb, wqr[...], dn, preferred_element_type=f32)
             + bqr[...]) * 0.25
  kr[...] = lax.dot_general(xb, wkr[...], dn, preferred_element_type=f32) + bkr[...]
  vr[...] = lax.dot_general(xb, wvr[...], dn, preferred_element_type=f32) + bvr[...]


def _qkv(x, Wq, bq, Wk, bk, Wv, bv):
  blk = 2000
  xspec = pl.BlockSpec((blk, HD), lambda i: (i, 0))
  wspec = pl.BlockSpec((HD, HD), lambda i: (0, 0))
  bspec = pl.BlockSpec((1, HD), lambda i: (0, 0))
  ospec = pl.BlockSpec((blk, HD), lambda i: (i, 0))
  return pl.pallas_call(
      _qkv_body,
      grid=(N // blk,),
      in_specs=[xspec, wspec, bspec, wspec, bspec, wspec, bspec],
      out_specs=[ospec, ospec, ospec],
      out_shape=[jax.ShapeDtypeStruct((N, HD), f32)] * 3,
  )(x, Wq, bq.reshape(1, HD), Wk, bk.reshape(1, HD), Wv, bv.reshape(1, HD))


def _score_body(qs, kh, src, dst, score_o, tmax_o,
                eidx, kq, sb, tm_v, sems):
  c = lax.axis_index("c")
  s = lax.axis_index("s")
  wid = c * 16 + s
  base0 = wid * E_PER
  lane = lax.iota(i32, 16)
  masks = [lane == h for h in range(H)]

  def issue(p, ci):
    base = base0 + ci * B
    pltpu.sync_copy(src.at[pl.ds(base, B)], eidx.at[2 * p])
    pltpu.sync_copy(dst.at[pl.ds(base, B)], eidx.at[2 * p + 1])
    pltpu.async_copy(kh.at[eidx.at[2 * p]], kq.at[2 * p], sems.at[p])
    pltpu.async_copy(qs.at[eidx.at[2 * p + 1]], kq.at[2 * p + 1], sems.at[p])

  def wait(p):
    pltpu.make_async_copy(kh.at[eidx.at[2 * p]], kq.at[2 * p], sems.at[p]).wait()
    pltpu.make_async_copy(qs.at[eidx.at[2 * p + 1]], kq.at[2 * p + 1],
                          sems.at[p]).wait()

  issue(0, 0)
  issue(1, 1)

  def pair(k, rm):
    for p in range(2):
      ci = 2 * k + p
      wait(p)

      def edge(e, rm):
        row = jnp.zeros((16,), f32)
        for h in range(H):
          kv = kq[2 * p, e, pl.ds(h * D, D)]
          qv = kq[2 * p + 1, e, pl.ds(h * D, D)]
          sh = jnp.sum(kv * qv)
          row = jnp.where(masks[h], sh, row)
        sb[p, e, :] = row
        return jnp.maximum(rm, row)

      rm = lax.fori_loop(0, B, edge, rm)

      @pl.when(ci + 2 < NCH)
      def _():
        issue(p, ci + 2)

      pltpu.sync_copy(sb.at[p], score_o.at[pl.ds(base0 + ci * B, B)])
    return rm

  rm = lax.fori_loop(0, NCH // 2, pair, jnp.full((16,), -jnp.inf, f32))
  for r in range(8):
    tm_v[r, :] = rm
  pltpu.sync_copy(tm_v, tmax_o.at[wid])


def _score_call(qs, kh, src, dst):
  return pl.kernel(
      _score_body,
      out_type=(jax.ShapeDtypeStruct((E, 16), f32),
                jax.ShapeDtypeStruct((NW, 8, 16), f32)),
      mesh=_mesh(),
      compiler_params=_SC_PARAMS,
      scratch_types=[
          pltpu.VMEM((4, B), i32),
          pltpu.VMEM((4, B, HD), f32),
          pltpu.VMEM((2, B, 16), f32),
          pltpu.VMEM((8, 16), f32),
          pltpu.SemaphoreType.DMA((2,)),
      ],
  )(qs, kh, src, dst)


def _global_max(tmax, tm_v):
  def mred(t, m):
    pltpu.sync_copy(tmax.at[t], tm_v)
    return jnp.maximum(m, tm_v[0, :])

  return lax.fori_loop(0, NW, mred, jnp.full((16,), -jnp.inf, f32))


def _agg_body(score, tmax, vh, src, dst, outv, outs,
              eidx, vgd, srd, idx_v, zbuf, sems, accv):
  c = lax.axis_index("c")
  s = lax.axis_index("s")
  wid = c * 16 + s
  base0 = wid * E_PER
  lane = lax.iota(i32, 16)
  masks = [lane == h for h in range(H)]
  off = s * SUB_ROWS

  def zero_zbuf(i, _):
    for j in range(HD // 16):
      zbuf[i, pl.ds(j * 16, 16)] = jnp.zeros((16,), f32)
    return 0

  def zero_acc():
    lax.fori_loop(0, ZROWS, zero_zbuf, 0)
    for j in range(SUB_ROWS // ZROWS):
      idx_v[...] = lane + (off + j * ZROWS)
      pltpu.sync_copy(zbuf, accv.at[idx_v])

  def dump_acc(out_ref):
    for j in range(SUB_ROWS // ZROWS):
      o2 = off + j * ZROWS
      idx_v[...] = lane + o2
      pltpu.sync_copy(accv.at[idx_v], zbuf)
      pltpu.sync_copy(zbuf, out_ref.at[c, pl.ds(o2, ZROWS)])

  zero_acc()
  plsc.subcore_barrier()

  def mred(t, m):
    pltpu.sync_copy(tmax.at[t], srd.at[0, pl.ds(0, 8)])
    return jnp.maximum(m, srd[0, 0, :])

  m = lax.fori_loop(0, NW, mred, jnp.full((16,), -jnp.inf, f32))

  def issue1(p, ci):
    base = base0 + ci * B
    pltpu.sync_copy(src.at[pl.ds(base, B)], eidx.at[2 * p])
    pltpu.sync_copy(dst.at[pl.ds(base, B)], eidx.at[2 * p + 1])
    pltpu.async_copy(vh.at[eidx.at[2 * p]], vgd.at[p], sems.at[p])
    pltpu.async_copy(score.at[pl.ds(base, B)], srd.at[p], sems.at[p])

  def wait1(p, ci):
    base = base0 + ci * B
    pltpu.make_async_copy(vh.at[eidx.at[2 * p]], vgd.at[p], sems.at[p]).wait()
    pltpu.make_async_copy(score.at[pl.ds(base, B)], srd.at[p], sems.at[p]).wait()

  issue1(0, 0)
  issue1(1, 1)

  def pair1(k, _):
    for p in range(2):
      ci = 2 * k + p
      wait1(p, ci)

      def edge(e, _):
        prow = jnp.exp(srd[p, e, :] - m)
        for h in range(H):
          ah = jnp.sum(jnp.where(masks[h], prow, 0.0))
          vgd[p, e, pl.ds(h * D, D)] = vgd[p, e, pl.ds(h * D, D)] * ah
        return 0

      lax.fori_loop(0, B, edge, 0)
      pltpu.sync_copy(vgd.at[p], accv.at[eidx.at[2 * p + 1]], add=True)

      @pl.when(ci + 2 < NCH)
      def _():
        issue1(p, ci + 2)
    return 0

  lax.fori_loop(0, NCH // 2, pair1, 0)
  plsc.subcore_barrier()
  dump_acc(outv)
  plsc.subcore_barrier()

  zero_acc()
  plsc.subcore_barrier()

  def zero_vg(i, _):
    for j in range(HD // 16):
      vgd[0, i, pl.ds(j * 16, 16)] = jnp.zeros((16,), f32)
    return 0

  lax.fori_loop(0, B, zero_vg, 0)

  def issue2(p, ci):
    base = base0 + ci * B
    pltpu.sync_copy(dst.at[pl.ds(base, B)], eidx.at[2 * p + 1])
    pltpu.async_copy(score.at[pl.ds(base, B)], srd.at[p], sems.at[p])

  def wait2(p, ci):
    base = base0 + ci * B
    pltpu.make_async_copy(score.at[pl.ds(base, B)], srd.at[p], sems.at[p]).wait()

  issue2(0, 0)
  issue2(1, 1)

  def pair2(k, _):
    for p in range(2):
      ci = 2 * k + p
      wait2(p, ci)

      def edge(e, _):
        vgd[0, e, pl.ds(0, 16)] = jnp.exp(srd[p, e, :] - m)
        return 0

      lax.fori_loop(0, B, edge, 0)
      pltpu.sync_copy(vgd.at[0], accv.at[eidx.at[2 * p + 1]], add=True)

      @pl.when(ci + 2 < NCH)
      def _():
        issue2(p, ci + 2)
    return 0

  lax.fori_loop(0, NCH // 2, pair2, 0)
  plsc.subcore_barrier()
  dump_acc(outs)


def _agg_call(score, tmax, vh, src, dst):
  return pl.kernel(
      _agg_body,
      out_type=(jax.ShapeDtypeStruct((2, N_PAD, HD), f32),
                jax.ShapeDtypeStruct((2, N_PAD, HD), f32)),
      mesh=_mesh(),
      compiler_params=_SC_PARAMS,
      scratch_types=[
          pltpu.VMEM((4, B), i32),
          pltpu.VMEM((2, B, HD), f32),
          pltpu.VMEM((2, B, 16), f32),
          pltpu.VMEM((ZROWS,), i32),
          pltpu.VMEM((ZROWS, HD), f32),
          pltpu.SemaphoreType.DMA((2,)),
          pltpu.VMEM_SHARED((N_PAD, HD), f32),
      ],
  )(score, tmax, vh, src, dst)


def _norm_body(v0r, v1r, s0r, s1r, orf):
  blk = orf.shape[0]
  ssum = s0r[0][:, :H] + s1r[0][:, :H] + 1e-16
  inv = 1.0 / ssum
  invb = lax.broadcast_in_dim(inv, (blk, H, D), (0, 1)).reshape(blk, HD)
  orf[...] = (v0r[0] + v1r[0]) * invb


def _normalize(outv, outs):
  blk = 2000
  v0spec = pl.BlockSpec((1, blk, HD), lambda i: (0, i, 0))
  v1spec = pl.BlockSpec((1, blk, HD), lambda i: (1, i, 0))
  ospec = pl.BlockSpec((blk, HD), lambda i: (i, 0))
  return pl.pallas_call(
      _norm_body,
      grid=(N // blk,),
      in_specs=[v0spec, v1spec, v0spec, v1spec],
      out_specs=ospec,
      out_shape=jax.ShapeDtypeStruct((N, HD), f32),
  )(outv, outv, outs, outs)


@jax.jit
def kernel(x, edge_index, Wq, bq, Wk, bk, Wv, bv):
  src = edge_index[0].astype(i32)
  dst = edge_index[1].astype(i32)
  qs, kh, vh = _qkv(x, Wq, bq, Wk, bk, Wv, bv)
  score, tmax = _score_call(qs, kh, src, dst)
  outv, outs = _agg_call(score, tmax, vh, src, dst)
  wv = _normalize(outv, outs)
  return wv.reshape(N, H, D)
